# baseline (speedup 1.0000x reference)
import functools

import jax
import jax.numpy as jnp
from jax import lax
from jax.experimental import pallas as pl
from jax.experimental.pallas import tpu as pltpu
from jax.experimental.pallas import tpu_sc as plsc

D = 64
EPS = 1e-5

GV = 128
GPC = 8
CHUNK = GV * GPC


def _relayout_sc(tt, v):
    vp = ((v + 127) // 128) * 128
    n_tiles = vp // 128
    info = plsc.get_sparse_core_info()
    nc, ns = info.num_cores, info.num_subcores
    nw = nc * ns

    mesh = plsc.VectorSubcoreMesh(core_axis_name="c", subcore_axis_name="s")

    @functools.partial(
        pl.kernel,
        mesh=mesh,
        out_type=jax.ShapeDtypeStruct((vp // 2, 2 * D), jnp.float32),
        compiler_params=pltpu.CompilerParams(needs_layout_passes=False),
        scratch_types=[
            pltpu.VMEM((4, D, 128), jnp.float32),
            pltpu.VMEM((4, D, 128), jnp.float32),
            pltpu.SemaphoreType.DMA((4,)),
            pltpu.SemaphoreType.DMA((4,)),
        ],
    )
    def k(tt_hbm, out_hbm, buf, obuf, isem, osem):
        wid = lax.axis_index("s") * nc + lax.axis_index("c")
        t0 = (n_tiles * wid) // nw
        t1 = (n_tiles * (wid + 1)) // nw
        lanes = lax.iota(jnp.int32, 16)
        ones = jnp.ones((16,), jnp.int32)

        def load_desc(t, b):
            return pltpu.make_async_copy(
                tt_hbm.at[:, pl.ds(pl.multiple_of(t * 128, 128), 128)],
                buf.at[b],
                isem.at[b],
            )

        def store_desc(t, b):
            return pltpu.make_async_copy(
                obuf.at[b],
                out_hbm.at[pl.ds(pl.multiple_of(t * 64, 8), D)],
                osem.at[b],
            )

        def transpose_tile(b):
            def jbody(j, col):
                q = col >> 1
                cbase = (col & 1) * D + lanes
                for g in range(4):
                    vals = plsc.load_gather(buf.at[b], [lanes + 16 * g, col])
                    plsc.store_scatter(obuf.at[b], [q, cbase + 16 * g], vals)
                return col + ones

            plsc.parallel_loop(
                0, 128, step=1, unroll=16, carry=jnp.zeros((16,), jnp.int32)
            )(jbody)

        nbuf = 4

        def group_body(i, carry):
            for b in range(nbuf):
                t = t0 + nbuf * i + b

                @pl.when(t < t1)
                def _():
                    @pl.when(t + nbuf - 1 < t1)
                    def _():
                        load_desc(t + nbuf - 1, (b + nbuf - 1) % nbuf).start()

                    load_desc(t, b).wait()

                    @pl.when(t - nbuf >= t0)
                    def _():
                        store_desc(t - nbuf, b).wait()

                    transpose_tile(b)
                    store_desc(t, b).start()

            return carry

        n_groups = ((n_tiles + nw - 1) // nw + nbuf) // nbuf
        for b in range(nbuf - 1):
            @pl.when(t0 + b < t1)
            def _():
                load_desc(t0 + b, b).start()

        lax.fori_loop(0, n_groups, group_body, 0)
        m = t1 - t0

        for b in range(nbuf):
            @pl.when(m >= b + 1)
            def _():
                store_desc(t0 + b, b).wait()

    return k(tt)


def _gather_sc(table_rows, idx):
    n = idx.shape[0]
    info = plsc.get_sparse_core_info()
    nc, ns = info.num_cores, info.num_subcores
    nw = nc * ns
    n_per_w = n // nw
    n_chunks = n_per_w // CHUNK
    assert n_per_w % CHUNK == 0

    mesh = plsc.VectorSubcoreMesh(core_axis_name="c", subcore_axis_name="s")

    @functools.partial(
        pl.kernel,
        mesh=mesh,
        out_type=jax.ShapeDtypeStruct((n, D), jnp.float32),
        compiler_params=pltpu.CompilerParams(use_tc_tiling_on_sc=False),
        scratch_types=[
            pltpu.VMEM((CHUNK,), jnp.int32),
            pltpu.VMEM((CHUNK, D), jnp.float32),
            pltpu.SemaphoreType.DMA,
        ],
    )
    def k(table_hbm, idx_hbm, out_hbm, idx_v, rows_v, gsem):
        wid = lax.axis_index("s") * nc + lax.axis_index("c")
        row_base = wid * n_per_w

        def chunk_body(i, carry):
            off = pl.multiple_of(row_base + i * CHUNK, CHUNK)
            pltpu.sync_copy(idx_hbm.at[pl.ds(off, CHUNK)], idx_v)
            for j in range(GPC):
                pltpu.async_copy(
                    table_hbm.at[idx_v.at[pl.ds(j * GV, GV)]],
                    rows_v.at[pl.ds(j * GV, GV)],
                    gsem,
                )
            for j in range(GPC):
                pltpu.make_async_copy(
                    table_hbm.at[idx_v.at[pl.ds(j * GV, GV)]],
                    rows_v.at[pl.ds(j * GV, GV)],
                    gsem,
                ).wait()
            pltpu.sync_copy(rows_v, out_hbm.at[pl.ds(off, CHUNK)])
            return carry

        lax.fori_loop(0, n_chunks, chunk_body, 0)

    return k(table_rows, idx)


def _ln_tc(rows2, gamma_t, beta_t, s, b):
    h = b // 2

    def body(r_ref, g_ref, b_ref, o_ref):
        w = jnp.transpose(r_ref[...])
        u = jnp.concatenate([w[:D, :], w[D:, :]], axis=1)
        mean = jnp.mean(u, axis=0, keepdims=True)
        c = u - mean
        var = jnp.mean(c * c, axis=0, keepdims=True)
        o_ref[...] = (c * lax.rsqrt(var + EPS) * g_ref[...] + b_ref[...])[None]

    return pl.pallas_call(
        body,
        grid=(s,),
        in_specs=[
            pl.BlockSpec((h, 2 * D), lambda i: (i, 0)),
            pl.BlockSpec((D, 1), lambda i: (0, 0)),
            pl.BlockSpec((D, 1), lambda i: (0, 0)),
        ],
        out_specs=pl.BlockSpec((1, D, b), lambda i: (i, 0, 0)),
        out_shape=jax.ShapeDtypeStruct((s, D, b), jnp.float32),
    )(rows2, gamma_t, beta_t)


def kernel(x, table, gamma, beta):
    b, s = x.shape
    v = table.shape[0]
    tt = jnp.transpose(table)
    t2 = _relayout_sc(tt, v)
    table_rows = t2.reshape(t2.shape[0] * 2, D)
    xt3 = jnp.transpose(x).reshape(s, 2, b // 2)
    idx = jnp.transpose(xt3, (0, 2, 1)).reshape(-1).astype(jnp.int32)
    rows = _gather_sc(table_rows, idx)
    rows2 = rows.reshape((b * s) // 2, 2 * D)
    y = _ln_tc(rows2, gamma.reshape(D, 1), beta.reshape(D, 1), s, b)
    return jnp.transpose(y, (2, 0, 1))


# speedup vs baseline: 1.5873x; 1.5873x over previous
import functools

import jax
import jax.numpy as jnp
from jax import lax
from jax.experimental import pallas as pl
from jax.experimental.pallas import tpu as pltpu
from jax.experimental.pallas import tpu_sc as plsc

D = 64
EPS = 1e-5

GV = 128
GPC = 8
CHUNK = GV * GPC


def _relayout_sc(tt, v):
    vp = ((v + 127) // 128) * 128
    n_tiles = vp // 128
    info = plsc.get_sparse_core_info()
    nc, ns = info.num_cores, info.num_subcores
    nw = nc * ns

    mesh = plsc.VectorSubcoreMesh(core_axis_name="c", subcore_axis_name="s")

    @functools.partial(
        pl.kernel,
        mesh=mesh,
        out_type=jax.ShapeDtypeStruct((vp // 2, 2 * D), jnp.float32),
        compiler_params=pltpu.CompilerParams(needs_layout_passes=False),
        scratch_types=[
            pltpu.VMEM((4, D, 128), jnp.float32),
            pltpu.VMEM((4, D, 128), jnp.float32),
            pltpu.SemaphoreType.DMA((4,)),
            pltpu.SemaphoreType.DMA((4,)),
        ],
    )
    def k(tt_hbm, out_hbm, buf, obuf, isem, osem):
        wid = lax.axis_index("s") * nc + lax.axis_index("c")
        t0 = (n_tiles * wid) // nw
        t1 = (n_tiles * (wid + 1)) // nw
        lanes = lax.iota(jnp.int32, 16)
        ones = jnp.ones((16,), jnp.int32)

        def load_desc(t, b):
            return pltpu.make_async_copy(
                tt_hbm.at[:, pl.ds(pl.multiple_of(t * 128, 128), 128)],
                buf.at[b],
                isem.at[b],
            )

        def store_desc(t, b):
            return pltpu.make_async_copy(
                obuf.at[b],
                out_hbm.at[pl.ds(pl.multiple_of(t * 64, 8), D)],
                osem.at[b],
            )

        def transpose_tile(b):
            def jbody(j, col):
                c = (col + lanes) & 127
                q = c >> 1
                cbase = (c & 1) * D
                for g in range(4):
                    d = lanes + 16 * g
                    vals = plsc.load_gather(buf.at[b], [d, c])
                    plsc.store_scatter(obuf.at[b], [q, cbase + d], vals)
                return col + ones

            plsc.parallel_loop(
                0, 128, step=1, unroll=8, carry=jnp.zeros((16,), jnp.int32)
            )(jbody)

        nbuf = 4

        def group_body(i, carry):
            for b in range(nbuf):
                t = t0 + nbuf * i + b

                @pl.when(t < t1)
                def _():
                    @pl.when(t + nbuf - 1 < t1)
                    def _():
                        load_desc(t + nbuf - 1, (b + nbuf - 1) % nbuf).start()

                    load_desc(t, b).wait()

                    @pl.when(t - nbuf >= t0)
                    def _():
                        store_desc(t - nbuf, b).wait()

                    transpose_tile(b)
                    store_desc(t, b).start()

            return carry

        n_groups = ((n_tiles + nw - 1) // nw + nbuf) // nbuf
        for b in range(nbuf - 1):
            @pl.when(t0 + b < t1)
            def _():
                load_desc(t0 + b, b).start()

        lax.fori_loop(0, n_groups, group_body, 0)
        m = t1 - t0

        for b in range(nbuf):
            @pl.when(m >= b + 1)
            def _():
                store_desc(t0 + b, b).wait()

    return k(tt)


def _gather_sc(table_rows, idx):
    n = idx.shape[0]
    info = plsc.get_sparse_core_info()
    nc, ns = info.num_cores, info.num_subcores
    nw = nc * ns
    n_per_w = n // nw
    n_chunks = n_per_w // CHUNK
    assert n_per_w % CHUNK == 0

    mesh = plsc.VectorSubcoreMesh(core_axis_name="c", subcore_axis_name="s")

    @functools.partial(
        pl.kernel,
        mesh=mesh,
        out_type=jax.ShapeDtypeStruct((n, D), jnp.float32),
        compiler_params=pltpu.CompilerParams(use_tc_tiling_on_sc=False),
        scratch_types=[
            pltpu.VMEM((CHUNK,), jnp.int32),
            pltpu.VMEM((CHUNK, D), jnp.float32),
            pltpu.SemaphoreType.DMA,
        ],
    )
    def k(table_hbm, idx_hbm, out_hbm, idx_v, rows_v, gsem):
        wid = lax.axis_index("s") * nc + lax.axis_index("c")
        row_base = wid * n_per_w

        def chunk_body(i, carry):
            off = pl.multiple_of(row_base + i * CHUNK, CHUNK)
            pltpu.sync_copy(idx_hbm.at[pl.ds(off, CHUNK)], idx_v)
            for j in range(GPC):
                pltpu.async_copy(
                    table_hbm.at[idx_v.at[pl.ds(j * GV, GV)]],
                    rows_v.at[pl.ds(j * GV, GV)],
                    gsem,
                )
            for j in range(GPC):
                pltpu.make_async_copy(
                    table_hbm.at[idx_v.at[pl.ds(j * GV, GV)]],
                    rows_v.at[pl.ds(j * GV, GV)],
                    gsem,
                ).wait()
            pltpu.sync_copy(rows_v, out_hbm.at[pl.ds(off, CHUNK)])
            return carry

        lax.fori_loop(0, n_chunks, chunk_body, 0)

    return k(table_rows, idx)


def _ln_tc(rows2, gamma_t, beta_t, s, b):
    h = b // 2

    def body(r_ref, g_ref, b_ref, o_ref):
        w = jnp.transpose(r_ref[...])
        u = jnp.concatenate([w[:D, :], w[D:, :]], axis=1)
        mean = jnp.mean(u, axis=0, keepdims=True)
        c = u - mean
        var = jnp.mean(c * c, axis=0, keepdims=True)
        o_ref[...] = (c * lax.rsqrt(var + EPS) * g_ref[...] + b_ref[...])[None]

    return pl.pallas_call(
        body,
        grid=(s,),
        in_specs=[
            pl.BlockSpec((h, 2 * D), lambda i: (i, 0)),
            pl.BlockSpec((D, 1), lambda i: (0, 0)),
            pl.BlockSpec((D, 1), lambda i: (0, 0)),
        ],
        out_specs=pl.BlockSpec((1, D, b), lambda i: (i, 0, 0)),
        out_shape=jax.ShapeDtypeStruct((s, D, b), jnp.float32),
    )(rows2, gamma_t, beta_t)


def kernel(x, table, gamma, beta):
    b, s = x.shape
    v = table.shape[0]
    tt = jnp.transpose(table)
    t2 = _relayout_sc(tt, v)
    table_rows = t2.reshape(t2.shape[0] * 2, D)
    xt3 = jnp.transpose(x).reshape(s, 2, b // 2)
    idx = jnp.transpose(xt3, (0, 2, 1)).reshape(-1).astype(jnp.int32)
    rows = _gather_sc(table_rows, idx)
    rows2 = rows.reshape((b * s) // 2, 2 * D)
    y = _ln_tc(rows2, gamma.reshape(D, 1), beta.reshape(D, 1), s, b)
    return jnp.transpose(y, (2, 0, 1))


# speedup vs baseline: 1.6039x; 1.0105x over previous
import functools

import jax
import jax.numpy as jnp
from jax import lax
from jax.experimental import pallas as pl
from jax.experimental.pallas import tpu as pltpu
from jax.experimental.pallas import tpu_sc as plsc

D = 64
EPS = 1e-5

GV = 128
GPC = 4
CHUNK = GV * GPC


def _relayout_sc(tt, v):
    vp = ((v + 127) // 128) * 128
    n_tiles = vp // 128
    info = plsc.get_sparse_core_info()
    nc, ns = info.num_cores, info.num_subcores
    nw = nc * ns

    mesh = plsc.VectorSubcoreMesh(core_axis_name="c", subcore_axis_name="s")

    @functools.partial(
        pl.kernel,
        mesh=mesh,
        out_type=jax.ShapeDtypeStruct((vp // 2, 2 * D), jnp.float32),
        compiler_params=pltpu.CompilerParams(needs_layout_passes=False),
        scratch_types=[
            pltpu.VMEM((4, D, 128), jnp.float32),
            pltpu.VMEM((4, D, 128), jnp.float32),
            pltpu.SemaphoreType.DMA((4,)),
            pltpu.SemaphoreType.DMA((4,)),
        ],
    )
    def k(tt_hbm, out_hbm, buf, obuf, isem, osem):
        wid = lax.axis_index("s") * nc + lax.axis_index("c")
        t0 = (n_tiles * wid) // nw
        t1 = (n_tiles * (wid + 1)) // nw
        lanes = lax.iota(jnp.int32, 16)
        ones = jnp.ones((16,), jnp.int32)

        def load_desc(t, b):
            return pltpu.make_async_copy(
                tt_hbm.at[:, pl.ds(pl.multiple_of(t * 128, 128), 128)],
                buf.at[b],
                isem.at[b],
            )

        def store_desc(t, b):
            return pltpu.make_async_copy(
                obuf.at[b],
                out_hbm.at[pl.ds(pl.multiple_of(t * 64, 8), D)],
                osem.at[b],
            )

        def transpose_tile(b):
            def jbody(j, col):
                c = (col + lanes) & 127
                q = c >> 1
                cbase = (c & 1) * D
                for g in range(4):
                    d = lanes + 16 * g
                    vals = plsc.load_gather(buf.at[b], [d, c])
                    plsc.store_scatter(obuf.at[b], [q, cbase + d], vals)
                return col + ones

            plsc.parallel_loop(
                0, 128, step=1, unroll=8, carry=jnp.zeros((16,), jnp.int32)
            )(jbody)

        nbuf = 4

        def group_body(i, carry):
            for b in range(nbuf):
                t = t0 + nbuf * i + b

                @pl.when(t < t1)
                def _():
                    @pl.when(t + nbuf - 1 < t1)
                    def _():
                        load_desc(t + nbuf - 1, (b + nbuf - 1) % nbuf).start()

                    load_desc(t, b).wait()

                    @pl.when(t - nbuf >= t0)
                    def _():
                        store_desc(t - nbuf, b).wait()

                    transpose_tile(b)
                    store_desc(t, b).start()

            return carry

        n_groups = ((n_tiles + nw - 1) // nw + nbuf) // nbuf
        for b in range(nbuf - 1):
            @pl.when(t0 + b < t1)
            def _():
                load_desc(t0 + b, b).start()

        lax.fori_loop(0, n_groups, group_body, 0)
        m = t1 - t0

        for b in range(nbuf):
            @pl.when(m >= b + 1)
            def _():
                store_desc(t0 + b, b).wait()

    return k(tt)


def _gather_sc(table_rows, idx):
    n = idx.shape[0]
    info = plsc.get_sparse_core_info()
    nc, ns = info.num_cores, info.num_subcores
    nw = nc * ns
    n_per_w = n // nw
    n_chunks = n_per_w // CHUNK
    assert n_per_w % CHUNK == 0

    mesh = plsc.VectorSubcoreMesh(core_axis_name="c", subcore_axis_name="s")

    @functools.partial(
        pl.kernel,
        mesh=mesh,
        out_type=jax.ShapeDtypeStruct((n, D), jnp.float32),
        compiler_params=pltpu.CompilerParams(use_tc_tiling_on_sc=False),
        scratch_types=[
            pltpu.VMEM((2, CHUNK), jnp.int32),
            pltpu.VMEM((2, CHUNK, D), jnp.float32),
            pltpu.SemaphoreType.DMA((2,)),
            pltpu.SemaphoreType.DMA((2,)),
        ],
    )
    def k(table_hbm, idx_hbm, out_hbm, idx_v, rows_v, gsem, osem):
        wid = lax.axis_index("s") * nc + lax.axis_index("c")
        row_base = wid * n_per_w

        def fire(i, b):
            off = pl.multiple_of(row_base + i * CHUNK, CHUNK)
            pltpu.sync_copy(idx_hbm.at[pl.ds(off, CHUNK)], idx_v.at[b])
            for j in range(GPC):
                pltpu.async_copy(
                    table_hbm.at[idx_v.at[b, pl.ds(j * GV, GV)]],
                    rows_v.at[b, pl.ds(j * GV, GV)],
                    gsem.at[b],
                )

        def drain(i, b):
            for j in range(GPC):
                pltpu.make_async_copy(
                    table_hbm.at[idx_v.at[b, pl.ds(j * GV, GV)]],
                    rows_v.at[b, pl.ds(j * GV, GV)],
                    gsem.at[b],
                ).wait()

        def store_desc(i, b):
            off = pl.multiple_of(row_base + i * CHUNK, CHUNK)
            return pltpu.make_async_copy(
                rows_v.at[b],
                out_hbm.at[pl.ds(off, CHUNK)],
                osem.at[b],
            )

        def pair_body(p, carry):
            for b in (0, 1):
                i = 2 * p + b

                @pl.when(i < n_chunks)
                def _():
                    @pl.when(i + 1 < n_chunks)
                    def _():
                        @pl.when(i - 1 >= 0)
                        def _():
                            store_desc(i - 1, 1 - b).wait()

                        fire(i + 1, 1 - b)

                    drain(i, b)
                    store_desc(i, b).start()

            return carry

        fire(0, 0)
        lax.fori_loop(0, (n_chunks + 2) // 2, pair_body, 0)
        store_desc(0, (n_chunks - 2) % 2).wait()
        store_desc(0, (n_chunks - 1) % 2).wait()

    return k(table_rows, idx)


def _ln_tc(rows2, gamma_t, beta_t, s, b):
    h = b // 2

    def body(r_ref, g_ref, b_ref, o_ref):
        w = jnp.transpose(r_ref[...])
        u = jnp.concatenate([w[:D, :], w[D:, :]], axis=1)
        mean = jnp.mean(u, axis=0, keepdims=True)
        c = u - mean
        var = jnp.mean(c * c, axis=0, keepdims=True)
        o_ref[...] = (c * lax.rsqrt(var + EPS) * g_ref[...] + b_ref[...])[None]

    return pl.pallas_call(
        body,
        grid=(s,),
        in_specs=[
            pl.BlockSpec((h, 2 * D), lambda i: (i, 0)),
            pl.BlockSpec((D, 1), lambda i: (0, 0)),
            pl.BlockSpec((D, 1), lambda i: (0, 0)),
        ],
        out_specs=pl.BlockSpec((1, D, b), lambda i: (i, 0, 0)),
        out_shape=jax.ShapeDtypeStruct((s, D, b), jnp.float32),
    )(rows2, gamma_t, beta_t)


def kernel(x, table, gamma, beta):
    b, s = x.shape
    v = table.shape[0]
    tt = jnp.transpose(table)
    t2 = _relayout_sc(tt, v)
    table_rows = t2.reshape(t2.shape[0] * 2, D)
    xt3 = jnp.transpose(x).reshape(s, 2, b // 2)
    idx = jnp.transpose(xt3, (0, 2, 1)).reshape(-1).astype(jnp.int32)
    rows = _gather_sc(table_rows, idx)
    rows2 = rows.reshape((b * s) // 2, 2 * D)
    y = _ln_tc(rows2, gamma.reshape(D, 1), beta.reshape(D, 1), s, b)
    return jnp.transpose(y, (2, 0, 1))
